# baseline (device time: 28572 ns/iter reference)
import jax
import jax.numpy as jnp
from jax import lax
from jax.experimental import pallas as pl
from jax.experimental.pallas import tpu as pltpu

N_DEV = 16
B, SQ, DMODEL = 2, 256, 512
HQ_LOCAL, DH = 4, 64
ROWS = B * SQ
CHUNK = ROWS // N_DEV
WINDOW = 128


def _body(x_ref, wq_ref, k_ref, v_ref, wo_ref, out_ref,
          ctx_ref, part_ref, recv_ref, stage_ref, recv2_ref,
          send1, recv1, send2, recv2):
    my = lax.axis_index("i")

    barrier = pltpu.get_barrier_semaphore()
    for o in range(1, N_DEV):
        pl.semaphore_signal(
            barrier, inc=1,
            device_id=((my + o) % N_DEV,),
            device_id_type=pl.DeviceIdType.MESH,
        )

    qi = lax.broadcasted_iota(jnp.int32, (SQ, SQ), 0)
    ki = lax.broadcasted_iota(jnp.int32, (SQ, SQ), 1)
    mask = jnp.abs(qi - ki) <= WINDOW

    def p1_rdma(o):
        dst = (my + o) % N_DEV
        return pltpu.make_async_remote_copy(
            src_ref=part_ref.at[pl.ds(dst * CHUNK, CHUNK), :],
            dst_ref=recv_ref.at[o - 1],
            send_sem=send1.at[o - 1],
            recv_sem=recv1.at[o - 1],
            device_id=(dst,),
            device_id_type=pl.DeviceIdType.MESH,
        )

    wq = wq_ref[:, :].astype(jnp.bfloat16)
    wo = wo_ref[:, :].astype(jnp.bfloat16)
    for b in range(B):
        xb = x_ref[pl.ds(b * SQ, SQ), :].astype(jnp.bfloat16)
        qb = lax.dot_general(xb, wq, (((1,), (0,)), ((), ())),
                             preferred_element_type=jnp.float32)
        qb = qb.astype(jnp.bfloat16)
        for h in range(HQ_LOCAL):
            qbh = qb[:, h * DH:(h + 1) * DH]
            kbh = k_ref[b, h].astype(jnp.bfloat16)
            vbh = v_ref[b, h].astype(jnp.bfloat16)
            s = lax.dot_general(qbh, kbh, (((1,), (1,)), ((), ())),
                                preferred_element_type=jnp.float32)
            s = jnp.where(mask, s * 0.125, -1e9)
            m = jnp.max(s, axis=1, keepdims=True)
            w = jnp.exp(s - m)
            w = w / jnp.sum(w, axis=1, keepdims=True)
            cbh = lax.dot_general(w.astype(jnp.bfloat16), vbh,
                                  (((1,), (0,)), ((), ())),
                                  preferred_element_type=jnp.float32)
            ctx_ref[pl.ds(b * SQ, SQ), pl.ds(h * DH, DH)] = (
                cbh.astype(jnp.bfloat16))
        cb = ctx_ref[pl.ds(b * SQ, SQ), :]
        part_ref[pl.ds(b * SQ, SQ), :] = lax.dot_general(
            cb, wo, (((1,), (0,)), ((), ())),
            preferred_element_type=jnp.float32).astype(jnp.bfloat16)

        if b == 0:
            pl.semaphore_wait(barrier, N_DEV - 1)
        lo, hi = b * (SQ // CHUNK), (b + 1) * (SQ // CHUNK)
        for o in range(1, N_DEV):
            dst = (my + o) % N_DEV
            @pl.when(jnp.logical_and(dst >= lo, dst < hi))
            def _():
                p1_rdma(o).start()

    p1 = [p1_rdma(o) for o in range(1, N_DEV)]

    red = part_ref[pl.ds(my * CHUNK, CHUNK), :].astype(jnp.float32)
    for o in range(1, N_DEV):
        wait = pltpu.make_async_remote_copy(
            src_ref=recv_ref.at[o - 1],
            dst_ref=recv_ref.at[o - 1],
            send_sem=send1.at[o - 1],
            recv_sem=recv1.at[o - 1],
            device_id=(my,),
            device_id_type=pl.DeviceIdType.MESH,
        )
        wait.wait_recv()
        red = red + recv_ref[o - 1].astype(jnp.float32)
    out_ref[pl.ds(my * CHUNK, CHUNK), :] = red
    stage_ref[:, :] = red.astype(jnp.bfloat16)

    p2 = []
    for o in range(1, N_DEV):
        dst = (my + o) % N_DEV
        rdma = pltpu.make_async_remote_copy(
            src_ref=stage_ref,
            dst_ref=recv2_ref.at[o - 1],
            send_sem=send2.at[o - 1],
            recv_sem=recv2.at[o - 1],
            device_id=(dst,),
            device_id_type=pl.DeviceIdType.MESH,
        )
        rdma.start()
        p2.append(rdma)

    for o in range(1, N_DEV):
        src = (my - o) % N_DEV
        wait = pltpu.make_async_remote_copy(
            src_ref=recv2_ref.at[o - 1],
            dst_ref=recv2_ref.at[o - 1],
            send_sem=send2.at[o - 1],
            recv_sem=recv2.at[o - 1],
            device_id=(my,),
            device_id_type=pl.DeviceIdType.MESH,
        )
        wait.wait_recv()
        out_ref[pl.ds(src * CHUNK, CHUNK), :] = (
            recv2_ref[o - 1].astype(jnp.float32))

    for rdma in p1:
        rdma.wait_send()
    for rdma in p2:
        rdma.wait_send()


def kernel(x, Wq, K_ext, V_ext, Wo):
    my = lax.axis_index("i")
    x2 = x.reshape(ROWS, DMODEL)
    ks = lax.dynamic_slice_in_dim(K_ext, my * HQ_LOCAL, HQ_LOCAL, axis=2)
    vs = lax.dynamic_slice_in_dim(V_ext, my * HQ_LOCAL, HQ_LOCAL, axis=2)
    ks = jnp.transpose(ks, (0, 2, 1, 3))
    vs = jnp.transpose(vs, (0, 2, 1, 3))

    out = pl.pallas_call(
        _body,
        out_shape=jax.ShapeDtypeStruct((ROWS, DMODEL), jnp.float32),
        in_specs=[pl.BlockSpec(memory_space=pltpu.VMEM)] * 5,
        out_specs=pl.BlockSpec(memory_space=pltpu.VMEM),
        scratch_shapes=[
            pltpu.VMEM((ROWS, HQ_LOCAL * DH), jnp.bfloat16),
            pltpu.VMEM((ROWS, DMODEL), jnp.bfloat16),
            pltpu.VMEM((N_DEV - 1, CHUNK, DMODEL), jnp.bfloat16),
            pltpu.VMEM((CHUNK, DMODEL), jnp.bfloat16),
            pltpu.VMEM((N_DEV - 1, CHUNK, DMODEL), jnp.bfloat16),
            pltpu.SemaphoreType.DMA((N_DEV - 1,)),
            pltpu.SemaphoreType.DMA((N_DEV - 1,)),
            pltpu.SemaphoreType.DMA((N_DEV - 1,)),
            pltpu.SemaphoreType.DMA((N_DEV - 1,)),
        ],
        compiler_params=pltpu.CompilerParams(collective_id=0),
    )(x2, Wq, ks, vs, Wo)
    return out.reshape(B, SQ, DMODEL)


# device time: 27237 ns/iter; 1.0490x vs baseline; 1.0490x over previous
import jax
import jax.numpy as jnp
from jax import lax
from jax.experimental import pallas as pl
from jax.experimental.pallas import tpu as pltpu

N_DEV = 16
B, SQ, DMODEL = 2, 256, 512
HQ_LOCAL, DH = 4, 64
ROWS = B * SQ
CHUNK = ROWS // N_DEV
WINDOW = 128


def _body(x_ref, wq_ref, k_ref, v_ref, wo_ref, out_ref,
          ctx_ref, part_ref, recv_ref, stage_ref, recv2_ref,
          send1, recv1, send2, recv2):
    my = lax.axis_index("i")

    barrier = pltpu.get_barrier_semaphore()
    for o in range(1, N_DEV):
        pl.semaphore_signal(
            barrier, inc=1,
            device_id=((my + o) % N_DEV,),
            device_id_type=pl.DeviceIdType.MESH,
        )

    qi = lax.broadcasted_iota(jnp.int32, (SQ, SQ), 0)
    ki = lax.broadcasted_iota(jnp.int32, (SQ, SQ), 1)
    mask = jnp.abs(qi - ki) <= WINDOW

    def p1_rdma(o):
        dst = (my + o) % N_DEV
        return pltpu.make_async_remote_copy(
            src_ref=part_ref.at[pl.ds(dst * CHUNK, CHUNK), :],
            dst_ref=recv_ref.at[o - 1],
            send_sem=send1.at[o - 1],
            recv_sem=recv1.at[o - 1],
            device_id=(dst,),
            device_id_type=pl.DeviceIdType.MESH,
        )

    wq = wq_ref[:, :].astype(jnp.bfloat16)
    wo = wo_ref[:, :].astype(jnp.bfloat16)
    for b in range(B):
        xb = x_ref[pl.ds(b * SQ, SQ), :].astype(jnp.bfloat16)
        qb = lax.dot_general(xb, wq, (((1,), (0,)), ((), ())),
                             preferred_element_type=jnp.float32)
        qb = qb.astype(jnp.bfloat16)
        for h in range(HQ_LOCAL):
            qbh = qb[:, h * DH:(h + 1) * DH]
            kbh = k_ref[b, :, h, :].astype(jnp.bfloat16)
            vbh = v_ref[b, :, h, :].astype(jnp.bfloat16)
            s = lax.dot_general(qbh, kbh, (((1,), (1,)), ((), ())),
                                preferred_element_type=jnp.float32)
            s = jnp.where(mask, s * 0.125, -1e9)
            m = jnp.max(s, axis=1, keepdims=True)
            w = jnp.exp(s - m)
            w = w / jnp.sum(w, axis=1, keepdims=True)
            cbh = lax.dot_general(w.astype(jnp.bfloat16), vbh,
                                  (((1,), (0,)), ((), ())),
                                  preferred_element_type=jnp.float32)
            ctx_ref[pl.ds(b * SQ, SQ), pl.ds(h * DH, DH)] = (
                cbh.astype(jnp.bfloat16))
        cb = ctx_ref[pl.ds(b * SQ, SQ), :]
        part_ref[pl.ds(b * SQ, SQ), :] = lax.dot_general(
            cb, wo, (((1,), (0,)), ((), ())),
            preferred_element_type=jnp.float32).astype(jnp.bfloat16)

    pl.semaphore_wait(barrier, N_DEV - 1)
    p1 = [p1_rdma(o) for o in range(1, N_DEV)]
    for rdma in p1:
        rdma.start()

    red = part_ref[pl.ds(my * CHUNK, CHUNK), :].astype(jnp.float32)
    for o in range(1, N_DEV):
        wait = pltpu.make_async_remote_copy(
            src_ref=recv_ref.at[o - 1],
            dst_ref=recv_ref.at[o - 1],
            send_sem=send1.at[o - 1],
            recv_sem=recv1.at[o - 1],
            device_id=(my,),
            device_id_type=pl.DeviceIdType.MESH,
        )
        wait.wait_recv()
        red = red + recv_ref[o - 1].astype(jnp.float32)
    out_ref[pl.ds(my * CHUNK, CHUNK), :] = red
    stage_ref[:, :] = red.astype(jnp.bfloat16)

    p2 = []
    for o in range(1, N_DEV):
        dst = (my + o) % N_DEV
        rdma = pltpu.make_async_remote_copy(
            src_ref=stage_ref,
            dst_ref=recv2_ref.at[o - 1],
            send_sem=send2.at[o - 1],
            recv_sem=recv2.at[o - 1],
            device_id=(dst,),
            device_id_type=pl.DeviceIdType.MESH,
        )
        rdma.start()
        p2.append(rdma)

    for o in range(1, N_DEV):
        src = (my - o) % N_DEV
        wait = pltpu.make_async_remote_copy(
            src_ref=recv2_ref.at[o - 1],
            dst_ref=recv2_ref.at[o - 1],
            send_sem=send2.at[o - 1],
            recv_sem=recv2.at[o - 1],
            device_id=(my,),
            device_id_type=pl.DeviceIdType.MESH,
        )
        wait.wait_recv()
        out_ref[pl.ds(src * CHUNK, CHUNK), :] = (
            recv2_ref[o - 1].astype(jnp.float32))

    for rdma in p1:
        rdma.wait_send()
    for rdma in p2:
        rdma.wait_send()


def kernel(x, Wq, K_ext, V_ext, Wo):
    my = lax.axis_index("i")
    x2 = x.reshape(ROWS, DMODEL)
    ks = lax.dynamic_slice_in_dim(K_ext, my * HQ_LOCAL, HQ_LOCAL, axis=2)
    vs = lax.dynamic_slice_in_dim(V_ext, my * HQ_LOCAL, HQ_LOCAL, axis=2)

    out = pl.pallas_call(
        _body,
        out_shape=jax.ShapeDtypeStruct((ROWS, DMODEL), jnp.float32),
        in_specs=[pl.BlockSpec(memory_space=pltpu.VMEM)] * 5,
        out_specs=pl.BlockSpec(memory_space=pltpu.VMEM),
        scratch_shapes=[
            pltpu.VMEM((ROWS, HQ_LOCAL * DH), jnp.bfloat16),
            pltpu.VMEM((ROWS, DMODEL), jnp.bfloat16),
            pltpu.VMEM((N_DEV - 1, CHUNK, DMODEL), jnp.bfloat16),
            pltpu.VMEM((CHUNK, DMODEL), jnp.bfloat16),
            pltpu.VMEM((N_DEV - 1, CHUNK, DMODEL), jnp.bfloat16),
            pltpu.SemaphoreType.DMA((N_DEV - 1,)),
            pltpu.SemaphoreType.DMA((N_DEV - 1,)),
            pltpu.SemaphoreType.DMA((N_DEV - 1,)),
            pltpu.SemaphoreType.DMA((N_DEV - 1,)),
        ],
        compiler_params=pltpu.CompilerParams(collective_id=0),
    )(x2, Wq, ks, vs, Wo)
    return out.reshape(B, SQ, DMODEL)


# device time: 25796 ns/iter; 1.1076x vs baseline; 1.0559x over previous
import jax
import jax.numpy as jnp
from jax import lax
from jax.experimental import pallas as pl
from jax.experimental.pallas import tpu as pltpu

N_DEV = 16
B, SQ, DMODEL = 2, 256, 512
HQ_LOCAL, DH = 4, 64
ROWS = B * SQ
CHUNK = ROWS // N_DEV
WINDOW = 128


def _body(x_ref, wq_ref, k_ref, v_ref, wo_ref, out_ref,
          part_ref, recv_ref, stage_ref, recv2_ref,
          send1, recv1, send2, recv2):
    my = lax.axis_index("i")

    barrier = pltpu.get_barrier_semaphore()
    for o in range(1, N_DEV):
        pl.semaphore_signal(
            barrier, inc=1,
            device_id=((my + o) % N_DEV,),
            device_id_type=pl.DeviceIdType.MESH,
        )

    qi = lax.broadcasted_iota(jnp.int32, (SQ, SQ), 0)
    ki = lax.broadcasted_iota(jnp.int32, (SQ, SQ), 1)
    mask = jnp.abs(qi - ki) <= WINDOW

    def p1_rdma(o):
        dst = (my + o) % N_DEV
        return pltpu.make_async_remote_copy(
            src_ref=part_ref.at[pl.ds(dst * CHUNK, CHUNK), :],
            dst_ref=recv_ref.at[o - 1],
            send_sem=send1.at[o - 1],
            recv_sem=recv1.at[o - 1],
            device_id=(dst,),
            device_id_type=pl.DeviceIdType.MESH,
        )

    wq = wq_ref[:, :].astype(jnp.bfloat16)
    wo = wo_ref[:, :].astype(jnp.bfloat16)
    for b in range(B):
        xb = x_ref[pl.ds(b * SQ, SQ), :].astype(jnp.bfloat16)
        qb = lax.dot_general(xb, wq, (((1,), (0,)), ((), ())),
                             preferred_element_type=jnp.float32)
        qb = qb.astype(jnp.bfloat16)
        cparts = []
        for h in range(HQ_LOCAL):
            qbh = qb[:, h * DH:(h + 1) * DH]
            kbh = k_ref[b, h].astype(jnp.bfloat16)
            vbh = v_ref[b, h].astype(jnp.bfloat16)
            s = lax.dot_general(qbh, kbh, (((1,), (1,)), ((), ())),
                                preferred_element_type=jnp.float32)
            w = jnp.exp(jnp.where(mask, s * 0.125, -30.0))
            w = w / jnp.sum(w, axis=1, keepdims=True)
            cbh = lax.dot_general(w.astype(jnp.bfloat16), vbh,
                                  (((1,), (0,)), ((), ())),
                                  preferred_element_type=jnp.float32)
            cparts.append(cbh.astype(jnp.bfloat16))
        cb = jnp.concatenate(cparts, axis=1)
        part_ref[pl.ds(b * SQ, SQ), :] = lax.dot_general(
            cb, wo, (((1,), (0,)), ((), ())),
            preferred_element_type=jnp.float32).astype(jnp.bfloat16)

    pl.semaphore_wait(barrier, N_DEV - 1)
    p1 = [p1_rdma(o) for o in range(1, N_DEV)]
    for rdma in p1:
        rdma.start()

    red = part_ref[pl.ds(my * CHUNK, CHUNK), :].astype(jnp.float32)
    for o in range(1, N_DEV):
        wait = pltpu.make_async_remote_copy(
            src_ref=recv_ref.at[o - 1],
            dst_ref=recv_ref.at[o - 1],
            send_sem=send1.at[o - 1],
            recv_sem=recv1.at[o - 1],
            device_id=(my,),
            device_id_type=pl.DeviceIdType.MESH,
        )
        wait.wait_recv()
        red = red + recv_ref[o - 1].astype(jnp.float32)
    out_ref[pl.ds(my * CHUNK, CHUNK), :] = red
    stage_ref[:, :] = red.astype(jnp.bfloat16)

    p2 = []
    for o in range(1, N_DEV):
        dst = (my + o) % N_DEV
        rdma = pltpu.make_async_remote_copy(
            src_ref=stage_ref,
            dst_ref=recv2_ref.at[o - 1],
            send_sem=send2.at[o - 1],
            recv_sem=recv2.at[o - 1],
            device_id=(dst,),
            device_id_type=pl.DeviceIdType.MESH,
        )
        rdma.start()
        p2.append(rdma)

    for o in range(1, N_DEV):
        src = (my - o) % N_DEV
        wait = pltpu.make_async_remote_copy(
            src_ref=recv2_ref.at[o - 1],
            dst_ref=recv2_ref.at[o - 1],
            send_sem=send2.at[o - 1],
            recv_sem=recv2.at[o - 1],
            device_id=(my,),
            device_id_type=pl.DeviceIdType.MESH,
        )
        wait.wait_recv()
        out_ref[pl.ds(src * CHUNK, CHUNK), :] = (
            recv2_ref[o - 1].astype(jnp.float32))

    for rdma in p1:
        rdma.wait_send()
    for rdma in p2:
        rdma.wait_send()


def kernel(x, Wq, K_ext, V_ext, Wo):
    my = lax.axis_index("i")
    x2 = x.reshape(ROWS, DMODEL)
    ks = lax.dynamic_slice_in_dim(K_ext, my * HQ_LOCAL, HQ_LOCAL, axis=2)
    vs = lax.dynamic_slice_in_dim(V_ext, my * HQ_LOCAL, HQ_LOCAL, axis=2)
    ks = jnp.transpose(ks, (0, 2, 1, 3))
    vs = jnp.transpose(vs, (0, 2, 1, 3))

    out = pl.pallas_call(
        _body,
        out_shape=jax.ShapeDtypeStruct((ROWS, DMODEL), jnp.float32),
        in_specs=[pl.BlockSpec(memory_space=pltpu.VMEM)] * 5,
        out_specs=pl.BlockSpec(memory_space=pltpu.VMEM),
        scratch_shapes=[
            pltpu.VMEM((ROWS, DMODEL), jnp.bfloat16),
            pltpu.VMEM((N_DEV - 1, CHUNK, DMODEL), jnp.bfloat16),
            pltpu.VMEM((CHUNK, DMODEL), jnp.bfloat16),
            pltpu.VMEM((N_DEV - 1, CHUNK, DMODEL), jnp.bfloat16),
            pltpu.SemaphoreType.DMA((N_DEV - 1,)),
            pltpu.SemaphoreType.DMA((N_DEV - 1,)),
            pltpu.SemaphoreType.DMA((N_DEV - 1,)),
            pltpu.SemaphoreType.DMA((N_DEV - 1,)),
        ],
        compiler_params=pltpu.CompilerParams(collective_id=0),
    )(x2, Wq, ks, vs, Wo)
    return out.reshape(B, SQ, DMODEL)
